# Initial kernel scaffold; baseline (speedup 1.0000x reference)
import jax
import jax.numpy as jnp
from jax.experimental import pallas as pl


def kernel(logits, target):
    raise NotImplementedError("write your pallas kernel here")


# speedup vs baseline: 68.2785x; 68.2785x over previous
import functools

import jax
import jax.numpy as jnp
from jax import lax
from jax.experimental import pallas as pl
from jax.experimental.pallas import tpu as pltpu
from jax.experimental.pallas import tpu_sc as plsc

NCLASS = 21
H = 512
W = 512
NPIX = H * W
NBATCH = 4
NPAIRS = NBATCH * NCLASS
B_BINS = 1024
SUBHIST = 2 * B_BINS
LANES = 16
HIST_WORDS = LANES * SUBHIST
NWORKERS = 32
CHUNK = 32768
NCHUNK = NPIX // CHUNK
ROWS_BLK = 64


def _binidx_body(lg_ref, tg_ref, out_ref):
    lg = lg_ref[0]
    tg = tg_ref[0]
    m = jnp.max(lg, axis=0)
    ex = jnp.exp(lg - m[None])
    p = ex * (1.0 / jnp.sum(ex, axis=0))[None]
    cls = lax.broadcasted_iota(jnp.int32, lg.shape, 0)
    fg = tg[None] == cls
    e = jnp.where(fg, 1.0 - p, p)
    binv = jnp.minimum((e * B_BINS).astype(jnp.int32), B_BINS - 1)
    lane = lax.broadcasted_iota(jnp.int32, lg.shape, 2) % LANES
    out_ref[0] = lane * SUBHIST + jnp.where(fg, B_BINS, 0) + binv


def _binidx(logits, target):
    grid = (NBATCH, H // ROWS_BLK)
    return pl.pallas_call(
        _binidx_body,
        grid=grid,
        in_specs=[
            pl.BlockSpec((1, NCLASS, ROWS_BLK, W), lambda b, r: (b, 0, r, 0)),
            pl.BlockSpec((1, ROWS_BLK, W), lambda b, r: (b, r, 0)),
        ],
        out_specs=pl.BlockSpec((1, NCLASS, ROWS_BLK, W), lambda b, r: (b, 0, r, 0)),
        out_shape=jax.ShapeDtypeStruct((NBATCH, NCLASS, H, W), jnp.int32),
    )(logits, target)


def _sc_hist_body(idx_hbm, out_hbm, buf0, buf1, hist, merged, sem0, sem1):
    wid = lax.axis_index("s") * 2 + lax.axis_index("c")
    zeros16 = jnp.zeros((LANES,), jnp.float32)
    ones16 = jnp.ones((LANES,), jnp.float32)
    bufs = (buf0, buf1)
    sems = (sem0, sem1)

    def process_pair(pair):
        def zero_body(i, c):
            hist[pl.ds(i * LANES, LANES)] = zeros16
            return c
        lax.fori_loop(0, HIST_WORDS // LANES, zero_body, 0, unroll=8)

        base = pair * NPIX
        pending = pltpu.async_copy(
            idx_hbm.at[pl.ds(base, CHUNK)], bufs[0], sems[0])
        for k in range(NCHUNK):
            pending.wait()
            if k + 1 < NCHUNK:
                nxt = pltpu.async_copy(
                    idx_hbm.at[pl.ds(base + (k + 1) * CHUNK, CHUNK)],
                    bufs[(k + 1) % 2], sems[(k + 1) % 2])
            cur = bufs[k % 2]

            def scat_body(i, c):
                idxv = cur[pl.ds(i * LANES, LANES)]
                plsc.addupdate_scatter(hist, [idxv], ones16)
                return c
            lax.fori_loop(0, CHUNK // LANES, scat_body, 0, unroll=8)
            if k + 1 < NCHUNK:
                pending = nxt

        def merge_body(j, c):
            acc = hist[pl.ds(j * LANES, LANES)]
            for l in range(1, LANES):
                acc = acc + hist[pl.ds(l * SUBHIST + j * LANES, LANES)]
            merged[pl.ds(j * LANES, LANES)] = acc
            return c
        lax.fori_loop(0, SUBHIST // LANES, merge_body, 0)
        pltpu.sync_copy(merged, out_hbm.at[pl.ds(pair * SUBHIST, SUBHIST)])

    for rep in range(3):
        pair = wid + rep * NWORKERS

        @pl.when(pair < NPAIRS)
        def _():
            process_pair(pair)


@functools.cache
def _sc_hist():
    return pl.kernel(
        _sc_hist_body,
        out_type=jax.ShapeDtypeStruct((NPAIRS * SUBHIST,), jnp.float32),
        mesh=plsc.VectorSubcoreMesh(core_axis_name="c", subcore_axis_name="s"),
        compiler_params=pltpu.CompilerParams(needs_layout_passes=False),
        scratch_types=[
            pltpu.VMEM((CHUNK,), jnp.int32),
            pltpu.VMEM((CHUNK,), jnp.int32),
            pltpu.VMEM((HIST_WORDS,), jnp.float32),
            pltpu.VMEM((SUBHIST,), jnp.float32),
            pltpu.SemaphoreType.DMA,
            pltpu.SemaphoreType.DMA,
        ],
    )


def _finalize_body(h_ref, out_ref):
    hist = h_ref[...]
    h_bg = hist[:, :B_BINS]
    h_fg = hist[:, B_BINS:]
    h_all = h_bg + h_fg
    r = lax.broadcasted_iota(jnp.int32, (B_BINS, B_BINS), 0)
    c = lax.broadcasted_iota(jnp.int32, (B_BINS, B_BINS), 1)
    tri = (r >= c).astype(jnp.float32)
    n_cnt = jnp.dot(h_all, tri, preferred_element_type=jnp.float32)
    f_cnt = jnp.dot(h_fg, tri, preferred_element_type=jnp.float32)
    g = f_cnt[:, 0:1]
    jac = 1.0 - (g - f_cnt) / jnp.maximum(g + n_cnt - f_cnt, 1.0)
    loss = (jnp.sum(jac, axis=1) - 0.5 * jac[:, 0]) * (1.0 / B_BINS)
    present = (g[:, 0] > 0.0).astype(jnp.float32)
    lossm = loss * present
    rowb = lax.broadcasted_iota(jnp.int32, (NPAIRS, NBATCH), 0) // NCLASS
    colb = lax.broadcasted_iota(jnp.int32, (NPAIRS, NBATCH), 1)
    onehot = (rowb == colb).astype(jnp.float32)
    lb = jnp.sum(lossm[:, None] * onehot, axis=0)
    cb = jnp.sum(present[:, None] * onehot, axis=0)
    batch = jnp.where(cb > 0.0, lb / jnp.maximum(cb, 1.0), 0.0)
    out_ref[...] = jnp.mean(batch)[None, None]


def _finalize(hist):
    return pl.pallas_call(
        _finalize_body,
        out_shape=jax.ShapeDtypeStruct((1, 1), jnp.float32),
    )(hist)


def kernel(logits, target):
    idx = _binidx(logits, target)
    hist = _sc_hist()(idx.reshape(NPAIRS * NPIX))
    out = _finalize(hist.reshape(NPAIRS, SUBHIST))
    return out[0, 0]


# speedup vs baseline: 82.1477x; 1.2031x over previous
import functools

import jax
import jax.numpy as jnp
from jax import lax
from jax.experimental import pallas as pl
from jax.experimental.pallas import tpu as pltpu
from jax.experimental.pallas import tpu_sc as plsc

NCLASS = 21
H = 512
W = 512
NPIX = H * W
NBATCH = 4
NPAIRS = NBATCH * NCLASS
B_BINS = 1024
SUBHIST = 2 * B_BINS
LANES = 16
HIST_WORDS = LANES * SUBHIST
NWORKERS = 32
CHUNK = 32768
NCHUNK = NPIX // CHUNK
ROWS_A = 64
ROWS_B = 128
OUT_ROWS = NPIX // 128


def _logz_body(lg_ref, q_ref):
    lg = lg_ref[0]
    m = jnp.max(lg, axis=0)
    s = jnp.sum(jnp.exp(lg - m[None]), axis=0)
    q_ref[0] = m + jnp.log(s)


def _logz(logits):
    return pl.pallas_call(
        _logz_body,
        grid=(NBATCH, H // ROWS_A),
        in_specs=[
            pl.BlockSpec((1, NCLASS, ROWS_A, W), lambda b, r: (b, 0, r, 0)),
        ],
        out_specs=pl.BlockSpec((1, ROWS_A, W), lambda b, r: (b, r, 0)),
        out_shape=jax.ShapeDtypeStruct((NBATCH, H, W), jnp.float32),
    )(logits)


def _binidx_body(lg_ref, q_ref, tg_ref, out_ref):
    c = pl.program_id(1)
    k = pl.program_id(2)
    l = lg_ref[0, 0]
    q = q_ref[0, pl.ds(k * ROWS_B, ROWS_B), :]
    t = tg_ref[0, pl.ds(k * ROWS_B, ROWS_B), :]
    p = jnp.exp(l - q)
    fg = t == c
    e = jnp.where(fg, 1.0 - p, p)
    binv = jnp.minimum((e * B_BINS).astype(jnp.int32), B_BINS - 1)
    lane = lax.broadcasted_iota(jnp.int32, l.shape, 1) % LANES
    idx = lane * SUBHIST + jnp.where(fg, B_BINS, 0) + binv
    for j in range(W // 128):
        out_ref[0, pl.ds(j * ROWS_B, ROWS_B), :] = idx[:, j * 128:(j + 1) * 128]


def _binidx(logits, q, target):
    blk_rows = ROWS_B * W // 128
    return pl.pallas_call(
        _binidx_body,
        grid=(NBATCH, NCLASS, H // ROWS_B),
        in_specs=[
            pl.BlockSpec((1, 1, ROWS_B, W), lambda b, c, k: (b, c, k, 0)),
            pl.BlockSpec((1, H, W), lambda b, c, k: (b, 0, 0)),
            pl.BlockSpec((1, H, W), lambda b, c, k: (b, 0, 0)),
        ],
        out_specs=pl.BlockSpec(
            (1, blk_rows, 128), lambda b, c, k: (c, b * (H // ROWS_B) + k, 0)),
        out_shape=jax.ShapeDtypeStruct(
            (NCLASS, NBATCH * OUT_ROWS, 128), jnp.int32),
    )(logits, q, target)


def _sc_hist_body(idx_hbm, out_hbm, buf0, buf1, hist, merged, sem0, sem1):
    wid = lax.axis_index("s") * 2 + lax.axis_index("c")
    zeros16 = jnp.zeros((LANES,), jnp.float32)
    ones16 = jnp.ones((LANES,), jnp.float32)
    bufs = (buf0, buf1)
    sems = (sem0, sem1)

    def process_pair(pair):
        @plsc.parallel_loop(0, HIST_WORDS // LANES, unroll=8)
        def _(i):
            hist[pl.ds(i * LANES, LANES)] = zeros16

        base = pair * NPIX
        pending = pltpu.async_copy(
            idx_hbm.at[pl.ds(base, CHUNK)], bufs[0], sems[0])
        for k in range(NCHUNK):
            pending.wait()
            if k + 1 < NCHUNK:
                nxt = pltpu.async_copy(
                    idx_hbm.at[pl.ds(base + (k + 1) * CHUNK, CHUNK)],
                    bufs[(k + 1) % 2], sems[(k + 1) % 2])
            cur = bufs[k % 2]

            @plsc.parallel_loop(0, CHUNK // LANES, unroll=8)
            def _(i):
                idxv = cur[pl.ds(i * LANES, LANES)]
                plsc.addupdate_scatter(hist, [idxv], ones16)

            if k + 1 < NCHUNK:
                pending = nxt

        @plsc.parallel_loop(0, SUBHIST // LANES, unroll=2)
        def _(j):
            acc = hist[pl.ds(j * LANES, LANES)]
            for l in range(1, LANES):
                acc = acc + hist[pl.ds(l * SUBHIST + j * LANES, LANES)]
            merged[pl.ds(j * LANES, LANES)] = acc

        pltpu.sync_copy(merged, out_hbm.at[pl.ds(pair * SUBHIST, SUBHIST)])

    for rep in range(3):
        pair = wid + rep * NWORKERS

        @pl.when(pair < NPAIRS)
        def _():
            process_pair(pair)


@functools.cache
def _sc_hist():
    return pl.kernel(
        _sc_hist_body,
        out_type=jax.ShapeDtypeStruct((NPAIRS * SUBHIST,), jnp.float32),
        mesh=plsc.VectorSubcoreMesh(core_axis_name="c", subcore_axis_name="s"),
        compiler_params=pltpu.CompilerParams(needs_layout_passes=False),
        scratch_types=[
            pltpu.VMEM((CHUNK,), jnp.int32),
            pltpu.VMEM((CHUNK,), jnp.int32),
            pltpu.VMEM((HIST_WORDS,), jnp.float32),
            pltpu.VMEM((SUBHIST,), jnp.float32),
            pltpu.SemaphoreType.DMA,
            pltpu.SemaphoreType.DMA,
        ],
    )


def _finalize_body(h_ref, out_ref):
    hist = h_ref[...]
    h_bg = hist[:, :B_BINS]
    h_fg = hist[:, B_BINS:]
    h_all = h_bg + h_fg
    r = lax.broadcasted_iota(jnp.int32, (B_BINS, B_BINS), 0)
    c = lax.broadcasted_iota(jnp.int32, (B_BINS, B_BINS), 1)
    tri = (r >= c).astype(jnp.float32)
    n_cnt = jnp.dot(h_all, tri, preferred_element_type=jnp.float32)
    f_cnt = jnp.dot(h_fg, tri, preferred_element_type=jnp.float32)
    g = f_cnt[:, 0:1]
    jac = 1.0 - (g - f_cnt) / jnp.maximum(g + n_cnt - f_cnt, 1.0)
    loss = (jnp.sum(jac, axis=1) - 0.5 * jac[:, 0]) * (1.0 / B_BINS)
    present = (g[:, 0] > 0.0).astype(jnp.float32)
    lossm = loss * present
    rowb = lax.broadcasted_iota(jnp.int32, (NPAIRS, NBATCH), 0) % NBATCH
    colb = lax.broadcasted_iota(jnp.int32, (NPAIRS, NBATCH), 1)
    onehot = (rowb == colb).astype(jnp.float32)
    lb = jnp.sum(lossm[:, None] * onehot, axis=0)
    cb = jnp.sum(present[:, None] * onehot, axis=0)
    batch = jnp.where(cb > 0.0, lb / jnp.maximum(cb, 1.0), 0.0)
    out_ref[...] = jnp.mean(batch)[None, None]


def _finalize(hist):
    return pl.pallas_call(
        _finalize_body,
        out_shape=jax.ShapeDtypeStruct((1, 1), jnp.float32),
    )(hist)


def kernel(logits, target):
    q = _logz(logits)
    idx = _binidx(logits, q, target)
    hist = _sc_hist()(idx.reshape(NPAIRS * NPIX))
    out = _finalize(hist.reshape(NPAIRS, SUBHIST))
    return out[0, 0]
